# baseline (speedup 1.0000x reference)
import functools

import jax
import jax.numpy as jnp
from jax import lax
from jax.experimental import pallas as pl
from jax.experimental.pallas import tpu as pltpu
from jax.experimental.pallas import tpu_sc as plsc

VOCAB = 1_000_000
D = 64
B = 4096 * 200

NC = 2
NS = 16
NW = NC * NS
ROWS_PER_W = B // NW
CHUNK = 512
NCHUNK = ROWS_PER_W // CHUNK


def _gather_body(ids_hbm, table_hbm, out_hbm, idx_v, rows_v, gsem):
  wid = lax.axis_index("s") * NC + lax.axis_index("c")
  base = wid * ROWS_PER_W

  def step(g, carry):
    off = base + g * CHUNK
    pltpu.sync_copy(ids_hbm.at[pl.ds(off, CHUNK)], idx_v)
    pltpu.async_copy(table_hbm.at[idx_v], rows_v, gsem).wait()
    pltpu.sync_copy(rows_v, out_hbm.at[pl.ds(off, CHUNK)])
    return carry

  lax.fori_loop(0, NCHUNK, step, 0)


@jax.jit
def _embed(ids_flat, weight):
  mesh = plsc.VectorSubcoreMesh(core_axis_name="c", subcore_axis_name="s")
  k = pl.kernel(
      _gather_body,
      out_type=jax.ShapeDtypeStruct((B, D), jnp.float32),
      mesh=mesh,
      scratch_types=[
          pltpu.VMEM((CHUNK,), jnp.int32),
          pltpu.VMEM((CHUNK, D), jnp.float32),
          pltpu.SemaphoreType.DMA,
      ],
      compiler_params=pltpu.CompilerParams(use_tc_tiling_on_sc=False),
  )
  return k(ids_flat, weight)


def kernel(input_ids, weight):
  ids_flat = input_ids.astype(jnp.int32).reshape(-1)
  out = _embed(ids_flat, weight)
  return out.reshape(input_ids.shape + (D,))


# speedup vs baseline: 1.0444x; 1.0444x over previous
import jax
import jax.numpy as jnp
from jax import lax
from jax.experimental import pallas as pl
from jax.experimental.pallas import tpu as pltpu
from jax.experimental.pallas import tpu_sc as plsc

VOCAB = 1_000_000
D = 64
B = 4096 * 200

NC = 2
NS = 16
NW = NC * NS
ROWS_PER_W = B // NW
CHUNK = 320
NCHUNK = ROWS_PER_W // CHUNK
NBUF = 4
NGRP = NCHUNK // NBUF


def _gather_body(ids_hbm, table_hbm, out_hbm, idx_all, rows, *sems):
  gsem = sems[:NBUF]
  ssem = sems[NBUF:]
  wid = lax.axis_index("s") * NC + lax.axis_index("c")
  base = wid * ROWS_PER_W
  pltpu.sync_copy(ids_hbm.at[pl.ds(base, ROWS_PER_W)], idx_all)

  def gather_desc(g, b):
    off = pl.multiple_of(g * CHUNK, 8)
    return pltpu.make_async_copy(
        table_hbm.at[idx_all.at[pl.ds(off, CHUNK)]], rows.at[b], gsem[b])

  def store_desc(g, b):
    off = pl.multiple_of(base + g * CHUNK, 8)
    return pltpu.make_async_copy(
        rows.at[b], out_hbm.at[pl.ds(off, CHUNK)], ssem[b])

  for b in range(NBUF):
    gather_desc(b, b).start()

  def group(i, carry):
    g0 = i * NBUF
    for b in range(NBUF):
      g = g0 + b
      gather_desc(g, b).wait()
      store_desc(g, b).start()
      store_desc(g, b).wait()
      gather_desc(g + NBUF, b).start()
    return carry

  lax.fori_loop(0, NGRP - 1, group, 0)

  g0 = (NGRP - 1) * NBUF
  for b in range(NBUF):
    g = g0 + b
    gather_desc(g, b).wait()
    store_desc(g, b).start()
  for b in range(NBUF):
    store_desc(g0 + b, b).wait()


@jax.jit
def _embed(ids_flat, weight):
  mesh = plsc.VectorSubcoreMesh(core_axis_name="c", subcore_axis_name="s")
  k = pl.kernel(
      _gather_body,
      out_type=jax.ShapeDtypeStruct((B, D), jnp.float32),
      mesh=mesh,
      scratch_types=[
          pltpu.VMEM((ROWS_PER_W,), jnp.int32),
          pltpu.VMEM((NBUF, CHUNK, D), jnp.float32),
      ] + [pltpu.SemaphoreType.DMA] * (2 * NBUF),
      compiler_params=pltpu.CompilerParams(use_tc_tiling_on_sc=False),
  )
  return k(ids_flat, weight)


def kernel(input_ids, weight):
  ids_flat = input_ids.astype(jnp.int32).reshape(-1)
  out = _embed(ids_flat, weight)
  return out.reshape(input_ids.shape + (D,))


# speedup vs baseline: 1.0489x; 1.0044x over previous
import jax
import jax.numpy as jnp
from jax import lax
from jax.experimental import pallas as pl
from jax.experimental.pallas import tpu as pltpu
from jax.experimental.pallas import tpu_sc as plsc

VOCAB = 1_000_000
D = 64
NB = 4096
NT = 200
B = NB * NT

NC = 2
NS = 16
NW = NC * NS
ROWS_PER_W = B // NW
BATCH_PER_W = NB // NW
NBUF = 4
NGRP = BATCH_PER_W // NBUF


def _gather_body(ids_hbm, table_hbm, out_hbm, idx_all, rows, *sems):
  gsem = sems[:NBUF]
  ssem = sems[NBUF:]
  wid = lax.axis_index("s") * NC + lax.axis_index("c")
  base = wid * ROWS_PER_W
  bbase = wid * BATCH_PER_W
  pltpu.sync_copy(ids_hbm.at[pl.ds(base, ROWS_PER_W)], idx_all)

  def gather_desc(g, b):
    off = pl.multiple_of(g * NT, 8)
    return pltpu.make_async_copy(
        table_hbm.at[idx_all.at[pl.ds(off, NT)]], rows.at[b], gsem[b])

  def store_desc(g, b):
    return pltpu.make_async_copy(rows.at[b], out_hbm.at[bbase + g], ssem[b])

  for b in range(NBUF):
    gather_desc(b, b).start()

  def group(i, carry):
    g0 = i * NBUF
    for b in range(NBUF):
      g = g0 + b
      gather_desc(g, b).wait()
      store_desc(g, b).start()
      store_desc(g, b).wait()
      gather_desc(g + NBUF, b).start()
    return carry

  lax.fori_loop(0, NGRP - 1, group, 0)

  g0 = (NGRP - 1) * NBUF
  for b in range(NBUF):
    g = g0 + b
    gather_desc(g, b).wait()
    store_desc(g, b).start()
  for b in range(NBUF):
    store_desc(g0 + b, b).wait()


@jax.jit
def _embed(ids_flat, weight):
  mesh = plsc.VectorSubcoreMesh(core_axis_name="c", subcore_axis_name="s")
  k = pl.kernel(
      _gather_body,
      out_type=jax.ShapeDtypeStruct((NB, NT, D), jnp.float32),
      mesh=mesh,
      scratch_types=[
          pltpu.VMEM((ROWS_PER_W,), jnp.int32),
          pltpu.VMEM((NBUF, NT, D), jnp.float32),
      ] + [pltpu.SemaphoreType.DMA] * (2 * NBUF),
      compiler_params=pltpu.CompilerParams(use_tc_tiling_on_sc=False),
  )
  return k(ids_flat, weight)


def kernel(input_ids, weight):
  ids_flat = input_ids.astype(jnp.int32).reshape(-1)
  return _embed(ids_flat, weight)


# speedup vs baseline: 1.3101x; 1.2490x over previous
import jax
import jax.numpy as jnp
from jax import lax
from jax.experimental import pallas as pl
from jax.experimental.pallas import tpu as pltpu
from jax.experimental.pallas import tpu_sc as plsc

VOCAB = 1_000_000
D = 64
DP = 128
NB = 4096
NT = 200

NC = 2
NS = 16
NW = NC * NS
BC = 128
NBUF = 4


def _body(ids_hbm, w_hbm, out_hbm, idx_v, rows_v, *sems):
  wid = lax.axis_index("s") * NC + lax.axis_index("c")
  b0 = wid * BC
  gsem = sems[:NBUF]
  ssem = sems[NBUF:]

  pltpu.sync_copy(ids_hbm.at[:, pl.ds(b0, BC)], idx_v)

  def gather_desc(u, b):
    return pltpu.make_async_copy(
        w_hbm.at[idx_v.at[u]], rows_v.at[b], gsem[b])

  def store_desc(u, b):
    return pltpu.make_async_copy(rows_v.at[b], out_hbm.at[u, wid], ssem[b])

  def unit(u, b):
    gather_desc(u, b).wait()
    store_desc(u, b).start()
    nxt = u + 2
    bn = (b + 2) % NBUF

    @pl.when(nxt < NT)
    def _():
      @pl.when(nxt >= NBUF)
      def _():
        store_desc(nxt - NBUF, bn).wait()

      gather_desc(nxt, bn).start()

  gather_desc(0, 0).start()
  gather_desc(1, 1).start()

  def group(g, carry):
    for i in range(NBUF):
      unit(g * NBUF + i, i)
    return carry

  lax.fori_loop(0, NT // NBUF, group, 0)

  for i in range(NBUF):
    store_desc(NT - NBUF + i, (NT - NBUF + i) % NBUF).wait()


@jax.jit
def _embed(ids_t, w_pad):
  mesh = plsc.VectorSubcoreMesh(core_axis_name="c", subcore_axis_name="s")
  k = pl.kernel(
      _body,
      out_type=jax.ShapeDtypeStruct((NT, NW, BC, DP), jnp.float32),
      mesh=mesh,
      scratch_types=[
          pltpu.VMEM((NT, BC), jnp.int32),
          pltpu.VMEM((NBUF, BC, DP), jnp.float32),
      ] + [pltpu.SemaphoreType.DMA] * (2 * NBUF),
  )
  return k(ids_t, w_pad)


def kernel(input_ids, weight):
  ids_t = jnp.transpose(input_ids.astype(jnp.int32), (1, 0))
  w_pad = jnp.pad(weight, ((0, 0), (0, DP - D)))
  o = _embed(ids_t, w_pad)
  return jnp.transpose(o[..., :D], (1, 2, 0, 3)).reshape(NB, NT, D)
